# baseline (speedup 1.0000x reference)
import functools

import jax
import jax.numpy as jnp
from jax.experimental import pallas as pl
from jax.experimental.pallas import tpu as pltpu

_MARGIN = 1.0
_SIM_MARGIN = 1.0 - _MARGIN / 2.0


def _terms(f, ag2, a2, sim, lab):
    f2 = jnp.sum(f * f, axis=1, keepdims=True)
    x2 = jax.lax.dot_general(
        f, ag2, (((1,), (1,)), ((), ())), preferred_element_type=jnp.float32
    )
    u = x2 - a2
    y = jnp.maximum(2048.0, u + ((_MARGIN + 2048.0) - f2))
    simmask = sim > _SIM_MARGIN
    if lab is not None:
        cols = jax.lax.broadcasted_iota(jnp.int32, sim.shape, 1)
        mask = simmask & (cols != lab)
        pos = jnp.sum(f2) - jnp.sum(jnp.where(cols == lab, u, 0.0))
    else:
        mask = simmask
        pos = 0.0
    packed = jnp.sum(jnp.where(mask, y, 0.0), axis=1)
    cnt = jnp.floor(packed * (1.0 / 2048.0))
    msum = packed - cnt * 2048.0
    has = cnt > 0.0
    mean_neg = jnp.where(has, msum / jnp.maximum(cnt, 1.0), 0.0)
    num = pos + jnp.sum(mean_neg)
    den = jnp.sum(jnp.where(has, 1.0, 0.0))
    if lab is not None:
        den = den + float(f.shape[0])
    return num, den


def _body(nsteps, ag_ref, af_ref, asim_ref, ay_ref, bf_ref, bsim_ref,
          out_ref, acc_ref):
    i = pl.program_id(0)

    @pl.when(i == 0)
    def _init():
        acc_ref[0] = 0.0
        acc_ref[1] = 0.0

    agents = ag_ref[...]
    a2 = jnp.sum(agents * agents, axis=1)[None, :]
    ag2 = agents + agents
    num_a, den_a = _terms(af_ref[...], ag2, a2, asim_ref[...], ay_ref[...])
    num_b, den_b = _terms(bf_ref[...], ag2, a2, bsim_ref[...], None)
    acc_ref[0] += num_a + num_b
    acc_ref[1] += den_a + den_b

    @pl.when(i == nsteps - 1)
    def _fin():
        out_ref[0, 0] = acc_ref[0] / acc_ref[1]


@jax.jit
def kernel(agents, a_f, a_sim, ay, b_f, b_sim):
    C, d = agents.shape
    Na = a_f.shape[0]
    BN = 256
    G = Na // BN
    ay2 = ay.astype(jnp.int32)[:, None]
    out = pl.pallas_call(
        functools.partial(_body, G),
        grid=(G,),
        in_specs=[
            pl.BlockSpec((C, d), lambda i: (0, 0)),
            pl.BlockSpec((BN, d), lambda i: (i, 0)),
            pl.BlockSpec((BN, C), lambda i: (i, 0)),
            pl.BlockSpec((BN, 1), lambda i: (i, 0)),
            pl.BlockSpec((BN, d), lambda i: (i, 0)),
            pl.BlockSpec((BN, C), lambda i: (i, 0)),
        ],
        out_specs=pl.BlockSpec(memory_space=pltpu.SMEM),
        out_shape=jax.ShapeDtypeStruct((1, 1), jnp.float32),
        scratch_shapes=[pltpu.SMEM((2,), jnp.float32)],
    )(agents, a_f, a_sim, ay2, b_f, b_sim)
    return out[0, 0]


# speedup vs baseline: 1.1719x; 1.1719x over previous
import functools

import jax
import jax.numpy as jnp
from jax.experimental import pallas as pl
from jax.experimental.pallas import tpu as pltpu

_MARGIN = 1.0
_SIM_MARGIN = 1.0 - _MARGIN / 2.0


def _terms(f, ag2, a2, sim, lab):
    f2 = jnp.sum(f * f, axis=1, keepdims=True)
    x2 = jax.lax.dot_general(
        f, ag2, (((1,), (1,)), ((), ())), preferred_element_type=jnp.float32
    )
    u = x2 - a2
    y = jnp.maximum(2048.0, u + ((_MARGIN + 2048.0) - f2))
    simmask = sim > _SIM_MARGIN
    if lab is not None:
        cols = jax.lax.broadcasted_iota(jnp.int32, sim.shape, 1)
        mask = simmask & (cols != lab)
        pos = jnp.sum(f2) - jnp.sum(jnp.where(cols == lab, u, 0.0))
    else:
        mask = simmask
        pos = 0.0
    packed = jnp.sum(jnp.where(mask, y, 0.0), axis=1)
    cnt = jnp.floor(packed * (1.0 / 2048.0))
    msum = packed - cnt * 2048.0
    has = cnt > 0.0
    mean_neg = jnp.where(has, msum / jnp.maximum(cnt, 1.0), 0.0)
    num = pos + jnp.sum(mean_neg)
    den = jnp.sum(jnp.where(has, 1.0, 0.0))
    if lab is not None:
        den = den + float(f.shape[0])
    return num, den


def _body(nsteps, ag_ref, af_ref, asim_ref, ay_ref, bf_ref, bsim_ref,
          out_ref, acc_ref):
    i = pl.program_id(0)

    @pl.when(i == 0)
    def _init():
        acc_ref[0] = 0.0
        acc_ref[1] = 0.0

    agents = ag_ref[...]
    a2 = jnp.sum(agents * agents, axis=1)[None, :]
    ag2 = agents + agents
    num_a, den_a = _terms(af_ref[...], ag2, a2, asim_ref[...], ay_ref[...])
    num_b, den_b = _terms(bf_ref[...], ag2, a2, bsim_ref[...], None)
    acc_ref[0] += num_a + num_b
    acc_ref[1] += den_a + den_b

    @pl.when(i == nsteps - 1)
    def _fin():
        out_ref[0, 0] = acc_ref[0] / acc_ref[1]


@jax.jit
def kernel(agents, a_f, a_sim, ay, b_f, b_sim):
    C, d = agents.shape
    Na = a_f.shape[0]
    BN = 1024
    G = Na // BN
    ay2 = ay.astype(jnp.int32)[:, None]
    out = pl.pallas_call(
        functools.partial(_body, G),
        grid=(G,),
        in_specs=[
            pl.BlockSpec((C, d), lambda i: (0, 0)),
            pl.BlockSpec((BN, d), lambda i: (i, 0)),
            pl.BlockSpec((BN, C), lambda i: (i, 0)),
            pl.BlockSpec((BN, 1), lambda i: (i, 0)),
            pl.BlockSpec((BN, d), lambda i: (i, 0)),
            pl.BlockSpec((BN, C), lambda i: (i, 0)),
        ],
        out_specs=pl.BlockSpec(memory_space=pltpu.SMEM),
        out_shape=jax.ShapeDtypeStruct((1, 1), jnp.float32),
        scratch_shapes=[pltpu.SMEM((2,), jnp.float32)],
    )(agents, a_f, a_sim, ay2, b_f, b_sim)
    return out[0, 0]
